# baseline (speedup 1.0000x reference)
import functools

import jax
import jax.numpy as jnp
from jax import lax
from jax.experimental import pallas as pl
from jax.experimental.pallas import tpu as pltpu
from jax.experimental.pallas import tpu_sc as plsc

_INFO = plsc.get_sparse_core_info()
_NC = _INFO.num_cores
_NS = _INFO.num_subcores
_L = _INFO.num_lanes
_NW = _NC * _NS

_K = 13
_D = 64
_P = 73
_C = 512
_NBUF = 2


def _do_group(in_b, tab_v, rows_b, g, iota):
    flat = g * (_L * _K) + iota * _K
    best_v = plsc.load_gather(in_b, [flat])
    best_i = jnp.zeros((_L,), jnp.int32)
    for k in range(1, _K):
        kv = jnp.full((_L,), k, jnp.int32)
        v = plsc.load_gather(in_b, [flat + k])
        upd = v > best_v
        best_v = jnp.where(upd, v, best_v)
        best_i = jnp.where(upd, kv, best_i)
    tbase = best_i * _P
    pos = g * _L + iota
    for j in range(_D):
        v = plsc.load_gather(tab_v, [tbase + j])
        plsc.store_scatter(rows_b, [pos, jnp.full((_L,), j, jnp.int32)], v)


def _body(x_hbm, tab_hbm, out_hbm, stg_v, tab_v, in_v, rows_v, sin, sout):
    wid = lax.axis_index("s") * _NC + lax.axis_index("c")
    n = out_hbm.shape[0]
    per_w = n // _NW
    chunks = per_w // _C
    w_base = wid * per_w
    iota = lax.iota(jnp.int32, _L)

    pltpu.sync_copy(tab_hbm, stg_v)
    for r in range(_K):
        for q in range(_D // _L):
            v = stg_v[pl.ds(r * _D + q * _L, _L)]
            plsc.store_scatter(tab_v, [r * _P + q * _L + iota], v)

    for b in range(_NBUF):
        pltpu.async_copy(
            x_hbm.at[pl.ds((w_base + b * _C) * _K, _C * _K)],
            in_v.at[b],
            sin.at[b],
        )

    def step(t2, _):
        for b in range(_NBUF):
            t = t2 * _NBUF + b
            base = pl.multiple_of(w_base + t * _C, _C)
            pltpu.make_async_copy(
                x_hbm.at[pl.ds(0, _C * _K)], in_v.at[b], sin.at[b]
            ).wait()
            @pl.when(t2 > 0)
            def _():
                pltpu.make_async_copy(
                    rows_v.at[b, :, pl.ds(0, _D)],
                    out_hbm.at[pl.ds(0, _C)],
                    sout.at[b],
                ).wait()

            @plsc.parallel_loop(0, _C // _L, 4, unroll=4)
            def group(g):
                for u in range(4):
                    _do_group(in_v.at[b], tab_v, rows_v.at[b], g + u,
                              iota)
            @pl.when(t + _NBUF < chunks)
            def _():
                pltpu.async_copy(
                    x_hbm.at[pl.ds((base + _NBUF * _C) * _K, _C * _K)],
                    in_v.at[b],
                    sin.at[b],
                )
            pltpu.async_copy(
                rows_v.at[b, :, pl.ds(0, _D)],
                out_hbm.at[pl.ds(base, _C)],
                sout.at[b],
            )
        return ()

    lax.fori_loop(0, chunks // _NBUF, step, (), unroll=False)
    for b in range(_NBUF):
        pltpu.make_async_copy(
            rows_v.at[b, :, pl.ds(0, _D)],
            out_hbm.at[pl.ds(0, _C)],
            sout.at[b],
        ).wait()


@functools.partial(jax.jit, static_argnames=())
def kernel(one_hot_pieces, piece_embedding):
    b = one_hot_pieces.shape[0]
    n = b * 64
    x = one_hot_pieces.reshape(n * _K)
    tab = piece_embedding.reshape(_K * _D)

    mesh = plsc.VectorSubcoreMesh(core_axis_name="c", subcore_axis_name="s")
    run = pl.kernel(
        _body,
        mesh=mesh,
        out_type=jax.ShapeDtypeStruct((n, _D), jnp.float32),
        scratch_types=[
            pltpu.VMEM((_K * _D,), jnp.float32),
            pltpu.VMEM((_K * _P,), jnp.float32),
            pltpu.VMEM((_NBUF, _C * _K), jnp.float32),
            pltpu.VMEM((_NBUF, _C, _P), jnp.float32),
            pltpu.SemaphoreType.DMA((_NBUF,)),
            pltpu.SemaphoreType.DMA((_NBUF,)),
        ],
        compiler_params=pltpu.CompilerParams(
            needs_layout_passes=False, use_tc_tiling_on_sc=False
        ),
    )
    out = run(x, tab)
    return out.reshape(b, 64, _D)


# speedup vs baseline: 1.3065x; 1.3065x over previous
import functools

import jax
import jax.numpy as jnp
from jax import lax
from jax.experimental import pallas as pl
from jax.experimental.pallas import tpu as pltpu
from jax.experimental.pallas import tpu_sc as plsc

_INFO = plsc.get_sparse_core_info()
_NC = _INFO.num_cores
_NS = _INFO.num_subcores
_L = _INFO.num_lanes
_NW = _NC * _NS

_K = 13
_D = 64
_P = 73
_C = 512
_NBUF = 2


def _do_group(in_b, tab_v, rows_b, g, iota):
    flat = g * (_L * _K) + iota * _K
    best_v = plsc.load_gather(in_b, [flat])
    best_i = jnp.zeros((_L,), jnp.int32)
    for k in range(1, _K):
        kv = jnp.full((_L,), k, jnp.int32)
        v = plsc.load_gather(in_b, [flat + k])
        upd = v > best_v
        best_v = jnp.where(upd, v, best_v)
        best_i = jnp.where(upd, kv, best_i)
    tbase = best_i * _P
    pos = g * _L + iota
    for j in range(_D):
        v = plsc.load_gather(tab_v, [tbase + j])
        plsc.store_scatter(rows_b, [pos, jnp.full((_L,), j, jnp.int32)], v)


def _body(x_hbm, tab_hbm, out_hbm, stg_v, tab_v, in_v, rows_v, sin, sout):
    wid = lax.axis_index("s") * _NC + lax.axis_index("c")
    n = out_hbm.shape[0]
    per_w = n // _NW
    chunks = per_w // _C
    w_base = wid * per_w
    iota = lax.iota(jnp.int32, _L)

    pltpu.sync_copy(tab_hbm, stg_v)
    for r in range(_K):
        for q in range(_D // _L):
            v = stg_v[pl.ds(r * _D + q * _L, _L)]
            plsc.store_scatter(tab_v, [r * _P + q * _L + iota], v)

    for b in range(_NBUF):
        pltpu.async_copy(
            x_hbm.at[pl.ds((w_base + b * _C) * _K, _C * _K)],
            in_v.at[b],
            sin.at[b],
        )

    def step(t2, _):
        for b in range(_NBUF):
            t = t2 * _NBUF + b
            base = pl.multiple_of(w_base + t * _C, _C)
            pltpu.make_async_copy(
                x_hbm.at[pl.ds(0, _C * _K)], in_v.at[b], sin.at[b]
            ).wait()
            @pl.when(t2 > 0)
            def _():
                pltpu.make_async_copy(
                    rows_v.at[b, :, pl.ds(0, _D)],
                    out_hbm.at[pl.ds(0, _C)],
                    sout.at[b],
                ).wait()

            @plsc.parallel_loop(0, _C // _L, 2, unroll=2)
            def group(g):
                for u in range(2):
                    _do_group(in_v.at[b], tab_v, rows_v.at[b], g + u,
                              iota)
            @pl.when(t + _NBUF < chunks)
            def _():
                pltpu.async_copy(
                    x_hbm.at[pl.ds((base + _NBUF * _C) * _K, _C * _K)],
                    in_v.at[b],
                    sin.at[b],
                )
            pltpu.async_copy(
                rows_v.at[b, :, pl.ds(0, _D)],
                out_hbm.at[pl.ds(base, _C)],
                sout.at[b],
            )
        return ()

    lax.fori_loop(0, chunks // _NBUF, step, (), unroll=False)
    for b in range(_NBUF):
        pltpu.make_async_copy(
            rows_v.at[b, :, pl.ds(0, _D)],
            out_hbm.at[pl.ds(0, _C)],
            sout.at[b],
        ).wait()


@functools.partial(jax.jit, static_argnames=())
def kernel(one_hot_pieces, piece_embedding):
    b = one_hot_pieces.shape[0]
    n = b * 64
    x = one_hot_pieces.reshape(n * _K)
    tab = piece_embedding.reshape(_K * _D)

    mesh = plsc.VectorSubcoreMesh(core_axis_name="c", subcore_axis_name="s")
    run = pl.kernel(
        _body,
        mesh=mesh,
        out_type=jax.ShapeDtypeStruct((n, _D), jnp.float32),
        scratch_types=[
            pltpu.VMEM((_K * _D,), jnp.float32),
            pltpu.VMEM((_K * _P,), jnp.float32),
            pltpu.VMEM((_NBUF, _C * _K), jnp.float32),
            pltpu.VMEM((_NBUF, _C, _P), jnp.float32),
            pltpu.SemaphoreType.DMA((_NBUF,)),
            pltpu.SemaphoreType.DMA((_NBUF,)),
        ],
        compiler_params=pltpu.CompilerParams(
            needs_layout_passes=False, use_tc_tiling_on_sc=False
        ),
    )
    out = run(x, tab)
    return out.reshape(b, 64, _D)


# speedup vs baseline: 1.3241x; 1.0135x over previous
import functools

import jax
import jax.numpy as jnp
from jax import lax
from jax.experimental import pallas as pl
from jax.experimental.pallas import tpu as pltpu
from jax.experimental.pallas import tpu_sc as plsc

_INFO = plsc.get_sparse_core_info()
_NC = _INFO.num_cores
_NS = _INFO.num_subcores
_L = _INFO.num_lanes
_NW = _NC * _NS

_K = 13
_D = 64
_P = 73
_C = 512
_NBUF = 2


def _do_group(in_b, tab_v, rows_b, g, iota):
    flat = g * (_L * _K) + iota * _K
    best_v = plsc.load_gather(in_b, [flat])
    best_i = jnp.zeros((_L,), jnp.int32)
    for k in range(1, _K):
        kv = jnp.full((_L,), k, jnp.int32)
        v = plsc.load_gather(in_b, [flat + k])
        upd = v > best_v
        best_v = jnp.where(upd, v, best_v)
        best_i = jnp.where(upd, kv, best_i)
    tbase = best_i * _P
    pos = g * _L + iota
    for j in range(_D):
        v = plsc.load_gather(tab_v, [tbase + j])
        plsc.store_scatter(rows_b, [pos, jnp.full((_L,), j, jnp.int32)], v)


def _body(x_hbm, tab_hbm, out_hbm, stg_v, tab_v, in_v, rows_v, sin, sout):
    wid = lax.axis_index("s") * _NC + lax.axis_index("c")
    n = out_hbm.shape[0]
    per_w = n // _NW
    chunks = per_w // _C
    w_base = wid * per_w
    iota = lax.iota(jnp.int32, _L)

    pltpu.sync_copy(tab_hbm, stg_v)
    for r in range(_K):
        for q in range(_D // _L):
            v = stg_v[pl.ds(r * _D + q * _L, _L)]
            plsc.store_scatter(tab_v, [r * _P + q * _L + iota], v)

    for b in range(_NBUF):
        pltpu.async_copy(
            x_hbm.at[pl.ds((w_base + b * _C) * _K, _C * _K)],
            in_v.at[b],
            sin.at[b],
        )

    def step(t2, _):
        for b in range(_NBUF):
            t = t2 * _NBUF + b
            base = pl.multiple_of(w_base + t * _C, _C)
            pltpu.make_async_copy(
                x_hbm.at[pl.ds(0, _C * _K)], in_v.at[b], sin.at[b]
            ).wait()
            @pl.when(t2 > 0)
            def _():
                for _h in range(2):
                    pltpu.make_async_copy(
                        rows_v.at[b, pl.ds(0, _C // 2), pl.ds(0, _D)],
                        out_hbm.at[pl.ds(0, _C // 2)],
                        sout.at[b],
                    ).wait()

            @plsc.parallel_loop(0, _C // _L // 2, 2, unroll=2)
            def group(g):
                for u in range(2):
                    _do_group(in_v.at[b], tab_v, rows_v.at[b], g + u,
                              iota)
            pltpu.async_copy(
                rows_v.at[b, pl.ds(0, _C // 2), pl.ds(0, _D)],
                out_hbm.at[pl.ds(base, _C // 2)],
                sout.at[b],
            )
            @plsc.parallel_loop(_C // _L // 2, _C // _L, 2, unroll=2)
            def group2(g):
                for u in range(2):
                    _do_group(in_v.at[b], tab_v, rows_v.at[b], g + u,
                              iota)
            @pl.when(t + _NBUF < chunks)
            def _():
                pltpu.async_copy(
                    x_hbm.at[pl.ds((base + _NBUF * _C) * _K, _C * _K)],
                    in_v.at[b],
                    sin.at[b],
                )
            pltpu.async_copy(
                rows_v.at[b, pl.ds(_C // 2, _C // 2), pl.ds(0, _D)],
                out_hbm.at[pl.ds(base + _C // 2, _C // 2)],
                sout.at[b],
            )
        return ()

    lax.fori_loop(0, chunks // _NBUF, step, (), unroll=False)
    for b in range(_NBUF):
        for _h in range(2):
            pltpu.make_async_copy(
                rows_v.at[b, pl.ds(0, _C // 2), pl.ds(0, _D)],
                out_hbm.at[pl.ds(0, _C // 2)],
                sout.at[b],
            ).wait()


@functools.partial(jax.jit, static_argnames=())
def kernel(one_hot_pieces, piece_embedding):
    b = one_hot_pieces.shape[0]
    n = b * 64
    x = one_hot_pieces.reshape(n * _K)
    tab = piece_embedding.reshape(_K * _D)

    mesh = plsc.VectorSubcoreMesh(core_axis_name="c", subcore_axis_name="s")
    run = pl.kernel(
        _body,
        mesh=mesh,
        out_type=jax.ShapeDtypeStruct((n, _D), jnp.float32),
        scratch_types=[
            pltpu.VMEM((_K * _D,), jnp.float32),
            pltpu.VMEM((_K * _P,), jnp.float32),
            pltpu.VMEM((_NBUF, _C * _K), jnp.float32),
            pltpu.VMEM((_NBUF, _C, _P), jnp.float32),
            pltpu.SemaphoreType.DMA((_NBUF,)),
            pltpu.SemaphoreType.DMA((_NBUF,)),
        ],
        compiler_params=pltpu.CompilerParams(
            needs_layout_passes=False, use_tc_tiling_on_sc=False
        ),
    )
    out = run(x, tab)
    return out.reshape(b, 64, _D)
